# baseline (speedup 1.0000x reference)
import functools

import jax
import jax.numpy as jnp
import numpy as np
from jax import lax
from jax.experimental import pallas as pl
from jax.experimental.pallas import tpu as pltpu
from jax.experimental.pallas import tpu_sc as plsc

NC = 2
NS = 16
NW = NC * NS
L = 16

N = 10000
E = 320000
D = 128
C = 500

CHUNK = 128
NB = 4
EPT = 10240
E_PAD = NW * EPT
NPT = 384
N_PAD = NW * NPT
DEG_P = 10240
CC = C * C
CC_P = 250880
R_P = 512

f32 = jnp.float32
i32 = jnp.int32


def _stage_a_body(src_h, dst_h, w_h, comms_h, x_h, ones_h,
                  deg_in_o, deg_out_o, coarse_o, embed_o, cnt_o, tot_o,
                  deg_in_sp, deg_out_sp, coarse_sp, embed_sp, cnt_sp, tot_sp,
                  comms_v, srcb, dstb, linb, wb, xb, cb, ob, db, tb,
                  zb, zrows, lsem, ssem):
    cid = lax.axis_index("c")
    sid = lax.axis_index("s")
    g = cid * NS + sid

    def zstep(i, carry):
        zb[pl.ds(i * L, L)] = jnp.zeros((L,), f32)
        return carry

    lax.fori_loop(jnp.int32(0), jnp.int32(15680 // L), zstep, jnp.int32(0))
    for r in range(32):
        for j in range(D // L):
            zrows[np.int32(r), pl.ds(j * L, L)] = jnp.zeros((L,), f32)

    pltpu.sync_copy(zb.at[pl.ds(0, 640)], deg_in_sp.at[pl.ds(sid * 640, 640)])
    pltpu.sync_copy(zb.at[pl.ds(0, 640)], deg_out_sp.at[pl.ds(sid * 640, 640)])
    pltpu.sync_copy(zb, coarse_sp.at[pl.ds(sid * 15680, 15680)])
    pltpu.sync_copy(zrows, embed_sp.at[pl.ds(sid * 32, 32), :])
    pltpu.sync_copy(zb.at[pl.ds(0, 32)], cnt_sp.at[pl.ds(sid * 32, 32)])
    pltpu.sync_copy(zb.at[pl.ds(0, 16)], tot_sp.at[pl.ds(sid * 16, 16)])
    pltpu.sync_copy(comms_h, comms_v)
    plsc.subcore_barrier()

    def edge_step(t, carry):
        base0 = g * EPT + t * (NB * CHUNK)
        descs = []
        for b in range(NB):
            base = base0 + b * CHUNK
            descs.append(pltpu.async_copy(
                src_h.at[pl.ds(base, CHUNK)], srcb.at[np.int32(b)], lsem))
            descs.append(pltpu.async_copy(
                dst_h.at[pl.ds(base, CHUNK)], dstb.at[np.int32(b)], lsem))
            descs.append(pltpu.async_copy(
                w_h.at[pl.ds(base, CHUNK)], wb.at[np.int32(b)], lsem))
        for d in descs:
            d.wait()
        for b in range(NB):
            for i in range(CHUNK // L):
                s16 = srcb[np.int32(b), pl.ds(i * L, L)]
                d16 = dstb[np.int32(b), pl.ds(i * L, L)]
                cs = plsc.load_gather(comms_v, [s16])
                cd = plsc.load_gather(comms_v, [d16])
                linb[np.int32(b), pl.ds(i * L, L)] = cs * C + cd
        sdescs = []
        for b in range(NB):
            sdescs.append(pltpu.async_copy(
                wb.at[np.int32(b)], deg_in_sp.at[dstb.at[np.int32(b)]],
                ssem, add=True))
            sdescs.append(pltpu.async_copy(
                wb.at[np.int32(b)], deg_out_sp.at[srcb.at[np.int32(b)]],
                ssem, add=True))
            sdescs.append(pltpu.async_copy(
                wb.at[np.int32(b)], coarse_sp.at[linb.at[np.int32(b)]],
                ssem, add=True))
        for d in sdescs:
            d.wait()
        return carry

    lax.fori_loop(jnp.int32(0), jnp.int32(EPT // (NB * CHUNK)), edge_step,
                  jnp.int32(0))

    for k in range(NPT // CHUNK):
        nbase = g * NPT + k * CHUNK
        pltpu.sync_copy(x_h.at[pl.ds(nbase, CHUNK), :], xb)
        pltpu.sync_copy(comms_h.at[pl.ds(nbase, CHUNK)], cb.at[np.int32(0)])
        pltpu.sync_copy(ones_h.at[pl.ds(nbase, CHUNK)], ob)
        pltpu.sync_copy(xb, embed_sp.at[cb.at[np.int32(0)]], add=True)
        pltpu.sync_copy(ob, cnt_sp.at[cb.at[np.int32(0)]], add=True)

    plsc.subcore_barrier()

    pltpu.sync_copy(deg_in_sp.at[pl.ds(sid * 640, 640)], db)
    acc = jnp.zeros((L,), f32)
    for i in range(640 // L):
        acc = acc + db[pl.ds(i * L, L)]
    tb[...] = acc
    pltpu.sync_copy(tb, tot_sp.at[pl.ds(sid * 16, 16)])
    plsc.subcore_barrier()

    pltpu.sync_copy(deg_in_sp.at[pl.ds(sid * 640, 640)], db)
    pltpu.sync_copy(db, deg_in_o.at[pl.ds(cid * DEG_P + sid * 640, 640)])
    pltpu.sync_copy(deg_out_sp.at[pl.ds(sid * 640, 640)], db)
    pltpu.sync_copy(db, deg_out_o.at[pl.ds(cid * DEG_P + sid * 640, 640)])
    pltpu.sync_copy(coarse_sp.at[pl.ds(sid * 15680, 15680)], zb)
    pltpu.sync_copy(zb, coarse_o.at[pl.ds(cid * CC_P + sid * 15680, 15680)])
    pltpu.sync_copy(embed_sp.at[pl.ds(sid * 32, 32), :], zrows)
    pltpu.sync_copy(zrows, embed_o.at[pl.ds(cid * R_P + sid * 32, 32), :])
    pltpu.sync_copy(cnt_sp.at[pl.ds(sid * 32, 32)], ob.at[pl.ds(0, 32)])
    pltpu.sync_copy(ob.at[pl.ds(0, 32)],
                    cnt_o.at[pl.ds(cid * R_P + sid * 32, 32)])
    pltpu.sync_copy(tot_sp.at[pl.ds(sid * 16, 16)], tb)
    pltpu.sync_copy(tb, tot_o.at[pl.ds(cid * 256 + sid * 16, 16)])


def _stage_b_body(deg_in_h, deg_out_h, coarse_h, embed_h, cnt_h, tot_h,
                  dist_o, coarse_o, embed_o,
                  a0, a1, b0, b1, dbuf, c0, c1, cbuf,
                  e0, e1, ebuf, t0, t1):
    cid = lax.axis_index("c")
    sid = lax.axis_index("s")
    g = cid * NS + sid

    pltpu.sync_copy(tot_h.at[pl.ds(0, 256)], t0)
    pltpu.sync_copy(tot_h.at[pl.ds(256, 256)], t1)
    acc = jnp.zeros((L,), f32)
    for i in range(256 // L):
        acc = acc + t0[pl.ds(i * L, L)] + t1[pl.ds(i * L, L)]
    inv2t = 0.5 / jnp.full((L,), jnp.sum(acc), f32)

    dbase = g * 320
    pltpu.sync_copy(deg_in_h.at[pl.ds(dbase, 320)], a0)
    pltpu.sync_copy(deg_in_h.at[pl.ds(DEG_P + dbase, 320)], a1)
    pltpu.sync_copy(deg_out_h.at[pl.ds(dbase, 320)], b0)
    pltpu.sync_copy(deg_out_h.at[pl.ds(DEG_P + dbase, 320)], b1)
    for i in range(320 // L):
        s = pl.ds(i * L, L)
        dbuf[s] = (a0[s] + a1[s] + b0[s] + b1[s]) * inv2t
    pltpu.sync_copy(dbuf, dist_o.at[pl.ds(dbase, 320)])

    for kchunk in range(5):
        cbase = g * 7840 + kchunk * 1568
        pltpu.sync_copy(coarse_h.at[pl.ds(cbase, 1568)], c0)
        pltpu.sync_copy(coarse_h.at[pl.ds(CC_P + cbase, 1568)], c1)
        for i in range(1568 // L):
            s = pl.ds(i * L, L)
            cbuf[s] = c0[s] + c1[s]
        pltpu.sync_copy(cbuf, coarse_o.at[pl.ds(cbase, 1568)])

    rbase = g * 16
    pltpu.sync_copy(embed_h.at[pl.ds(rbase, 16), :], e0)
    pltpu.sync_copy(embed_h.at[pl.ds(R_P + rbase, 16), :], e1)
    pltpu.sync_copy(cnt_h.at[pl.ds(rbase, 16)], t0.at[pl.ds(0, 16)])
    pltpu.sync_copy(cnt_h.at[pl.ds(R_P + rbase, 16)], t1.at[pl.ds(0, 16)])
    cnts_vec = t0[pl.ds(0, 16)] + t1[pl.ds(0, 16)]
    lanes = lax.iota(i32, L)
    for r in range(16):
        cr = jnp.sum(jnp.where(lanes == r, cnts_vec, 0.0))
        inv = 1.0 / jnp.maximum(jnp.full((L,), cr, f32), 1.0)
        for j in range(D // L):
            s = pl.ds(j * L, L)
            ebuf[np.int32(r), s] = (e0[np.int32(r), s] + e1[np.int32(r), s]) * inv
    pltpu.sync_copy(ebuf, embed_o.at[pl.ds(rbase, 16), :])


def _build_calls():
    mesh = plsc.VectorSubcoreMesh(core_axis_name="c", subcore_axis_name="s",
                                  num_cores=NC, num_subcores=NS)

    stage_a = functools.partial(
        pl.kernel,
        out_type=(
            jax.ShapeDtypeStruct((NC * DEG_P,), f32),
            jax.ShapeDtypeStruct((NC * DEG_P,), f32),
            jax.ShapeDtypeStruct((NC * CC_P,), f32),
            jax.ShapeDtypeStruct((NC * R_P, D), f32),
            jax.ShapeDtypeStruct((NC * R_P,), f32),
            jax.ShapeDtypeStruct((NC * 256,), f32),
        ),
        mesh=mesh,
        compiler_params=pltpu.CompilerParams(needs_layout_passes=False),
        scratch_types=[
            pltpu.VMEM_SHARED((DEG_P,), f32),
            pltpu.VMEM_SHARED((DEG_P,), f32),
            pltpu.VMEM_SHARED((CC_P,), f32),
            pltpu.VMEM_SHARED((R_P, D), f32),
            pltpu.VMEM_SHARED((R_P,), f32),
            pltpu.VMEM_SHARED((256,), f32),
            pltpu.VMEM((N_PAD,), i32),
            pltpu.VMEM((NB, CHUNK), i32),
            pltpu.VMEM((NB, CHUNK), i32),
            pltpu.VMEM((NB, CHUNK), i32),
            pltpu.VMEM((NB, CHUNK), f32),
            pltpu.VMEM((CHUNK, D), f32),
            pltpu.VMEM((1, CHUNK), i32),
            pltpu.VMEM((CHUNK,), f32),
            pltpu.VMEM((640,), f32),
            pltpu.VMEM((L,), f32),
            pltpu.VMEM((15680,), f32),
            pltpu.VMEM((32, D), f32),
            pltpu.SemaphoreType.DMA,
            pltpu.SemaphoreType.DMA,
        ],
    )(_stage_a_body)

    stage_b = functools.partial(
        pl.kernel,
        out_type=(
            jax.ShapeDtypeStruct((DEG_P,), f32),
            jax.ShapeDtypeStruct((CC_P,), f32),
            jax.ShapeDtypeStruct((R_P, D), f32),
        ),
        mesh=mesh,
        compiler_params=pltpu.CompilerParams(needs_layout_passes=False),
        scratch_types=[
            pltpu.VMEM((320,), f32),
            pltpu.VMEM((320,), f32),
            pltpu.VMEM((320,), f32),
            pltpu.VMEM((320,), f32),
            pltpu.VMEM((320,), f32),
            pltpu.VMEM((1568,), f32),
            pltpu.VMEM((1568,), f32),
            pltpu.VMEM((1568,), f32),
            pltpu.VMEM((16, D), f32),
            pltpu.VMEM((16, D), f32),
            pltpu.VMEM((16, D), f32),
            pltpu.VMEM((256,), f32),
            pltpu.VMEM((256,), f32),
        ],
    )(_stage_b_body)

    return stage_a, stage_b


_CALLS = []


@jax.jit
def kernel(x, edge_index, edge_weight, comms_idx):
    if not _CALLS:
        _CALLS.extend(_build_calls())
    _STAGE_A, _STAGE_B = _CALLS
    src = edge_index[0].astype(i32)
    dst = edge_index[1].astype(i32)
    w = edge_weight.astype(f32)
    comms = comms_idx.astype(i32)

    spread_e = (jnp.arange(E_PAD - E, dtype=i32) * 97) % N
    src_p = jnp.concatenate([src, spread_e])
    dst_p = jnp.concatenate([dst, spread_e])
    w_p = jnp.concatenate([w, jnp.zeros((E_PAD - E,), f32)])
    spread_n = (jnp.arange(N_PAD - N, dtype=i32) * 97) % C
    comms_p = jnp.concatenate([comms, spread_n])
    x_p = jnp.concatenate([x.astype(f32), jnp.zeros((N_PAD - N, D), f32)])
    ones_p = jnp.concatenate([jnp.ones((N,), f32), jnp.zeros((N_PAD - N,), f32)])

    deg_in2, deg_out2, coarse2, embed2, cnt2, tot2 = _STAGE_A(
        src_p, dst_p, w_p, comms_p, x_p, ones_p)
    dist_p, coarse_p, embed_p = _STAGE_B(
        deg_in2, deg_out2, coarse2, embed2, cnt2, tot2)

    embed = embed_p[:C]
    dist = dist_p[:N]
    coarse_w = coarse_p[:CC].reshape(C, C)
    return embed, dist, coarse_w


# speedup vs baseline: 1.1845x; 1.1845x over previous
import functools

import jax
import jax.numpy as jnp
import numpy as np
from jax import lax
from jax.experimental import pallas as pl
from jax.experimental.pallas import tpu as pltpu
from jax.experimental.pallas import tpu_sc as plsc

NC = 2
NS = 16
NW = NC * NS
L = 16

N = 10000
E = 320000
D = 128
C = 500

CHUNK = 128
NB = 4
EPT = 10240
E_PAD = NW * EPT
NPT = 384
N_PAD = NW * NPT
DEG_P = 10240
CC = C * C
CC_P = 250880
R_P = 512

f32 = jnp.float32
i32 = jnp.int32


def _stage_a_body(src_h, dst_h, w_h, comms_h, x_h, ones_h,
                  deg_in_o, deg_out_o, coarse_o, embed_o, cnt_o, tot_o,
                  deg_in_sp, deg_out_sp, coarse_sp, embed_sp, cnt_sp, tot_sp,
                  comms_v, srcb, dstb, linb, wb, xb, cb, ob, db, tb,
                  zb, zrows, lsem, ssem):
    cid = lax.axis_index("c")
    sid = lax.axis_index("s")
    g = cid * NS + sid

    def zstep(i, carry):
        zb[pl.ds(i * L, L)] = jnp.zeros((L,), f32)
        return carry

    lax.fori_loop(jnp.int32(0), jnp.int32(15680 // L), zstep, jnp.int32(0))
    for r in range(32):
        for j in range(D // L):
            zrows[np.int32(r), pl.ds(j * L, L)] = jnp.zeros((L,), f32)

    pltpu.sync_copy(zb.at[pl.ds(0, 640)], deg_in_sp.at[pl.ds(sid * 640, 640)])
    pltpu.sync_copy(zb.at[pl.ds(0, 640)], deg_out_sp.at[pl.ds(sid * 640, 640)])
    pltpu.sync_copy(zb, coarse_sp.at[pl.ds(sid * 15680, 15680)])
    pltpu.sync_copy(zrows, embed_sp.at[pl.ds(sid * 32, 32), :])
    pltpu.sync_copy(zb.at[pl.ds(0, 32)], cnt_sp.at[pl.ds(sid * 32, 32)])
    pltpu.sync_copy(zb.at[pl.ds(0, 16)], tot_sp.at[pl.ds(sid * 16, 16)])
    pltpu.sync_copy(comms_h, comms_v)
    plsc.subcore_barrier()

    def edge_step(t, carry):
        base0 = g * EPT + t * (NB * CHUNK)
        descs = []
        for b in range(NB):
            base = base0 + b * CHUNK
            descs.append(pltpu.async_copy(
                src_h.at[pl.ds(base, CHUNK)], srcb.at[np.int32(b)], lsem))
            descs.append(pltpu.async_copy(
                dst_h.at[pl.ds(base, CHUNK)], dstb.at[np.int32(b)], lsem))
            descs.append(pltpu.async_copy(
                w_h.at[pl.ds(base, CHUNK)], wb.at[np.int32(b)], lsem))
        for d in descs:
            d.wait()
        for b in range(NB):
            for i in range(CHUNK // L):
                s16 = srcb[np.int32(b), pl.ds(i * L, L)]
                d16 = dstb[np.int32(b), pl.ds(i * L, L)]
                cs = plsc.load_gather(comms_v, [s16])
                cd = plsc.load_gather(comms_v, [d16])
                linb[np.int32(b), pl.ds(i * L, L)] = cs * C + cd
        sdescs = []
        for b in range(NB):
            sdescs.append(pltpu.async_copy(
                wb.at[np.int32(b)], deg_in_sp.at[dstb.at[np.int32(b)]],
                ssem, add=True))
            sdescs.append(pltpu.async_copy(
                wb.at[np.int32(b)], deg_out_sp.at[srcb.at[np.int32(b)]],
                ssem, add=True))
            sdescs.append(pltpu.async_copy(
                wb.at[np.int32(b)], coarse_sp.at[linb.at[np.int32(b)]],
                ssem, add=True))
        for d in sdescs:
            d.wait()
        return carry

    lax.fori_loop(jnp.int32(0), jnp.int32(EPT // (NB * CHUNK)), edge_step,
                  jnp.int32(0))

    for k in range(NPT // CHUNK):
        nbase = g * NPT + k * CHUNK
        pltpu.sync_copy(x_h.at[pl.ds(nbase, CHUNK), :], xb)
        pltpu.sync_copy(comms_h.at[pl.ds(nbase, CHUNK)], cb.at[np.int32(0)])
        pltpu.sync_copy(ones_h.at[pl.ds(nbase, CHUNK)], ob)
        pltpu.sync_copy(xb, embed_sp.at[cb.at[np.int32(0)]], add=True)
        pltpu.sync_copy(ob, cnt_sp.at[cb.at[np.int32(0)]], add=True)

    plsc.subcore_barrier()

    pltpu.sync_copy(deg_in_sp.at[pl.ds(sid * 640, 640)], db)
    acc = jnp.zeros((L,), f32)
    for i in range(640 // L):
        acc = acc + db[pl.ds(i * L, L)]
    tb[...] = acc
    pltpu.sync_copy(tb, tot_sp.at[pl.ds(sid * 16, 16)])
    plsc.subcore_barrier()

    pltpu.sync_copy(deg_in_sp.at[pl.ds(sid * 640, 640)], db)
    pltpu.sync_copy(db, deg_in_o.at[pl.ds(cid * DEG_P + sid * 640, 640)])
    pltpu.sync_copy(deg_out_sp.at[pl.ds(sid * 640, 640)], db)
    pltpu.sync_copy(db, deg_out_o.at[pl.ds(cid * DEG_P + sid * 640, 640)])
    pltpu.sync_copy(coarse_sp.at[pl.ds(sid * 15680, 15680)], zb)
    pltpu.sync_copy(zb, coarse_o.at[pl.ds(cid * CC_P + sid * 15680, 15680)])
    pltpu.sync_copy(embed_sp.at[pl.ds(sid * 32, 32), :], zrows)
    pltpu.sync_copy(zrows, embed_o.at[pl.ds(cid * R_P + sid * 32, 32), :])
    pltpu.sync_copy(cnt_sp.at[pl.ds(sid * 32, 32)], ob.at[pl.ds(0, 32)])
    pltpu.sync_copy(ob.at[pl.ds(0, 32)],
                    cnt_o.at[pl.ds(cid * R_P + sid * 32, 32)])
    pltpu.sync_copy(tot_sp.at[pl.ds(sid * 16, 16)], tb)
    pltpu.sync_copy(tb, tot_o.at[pl.ds(cid * 256 + sid * 16, 16)])


def _stage_b_tc(di, do_, co, em, cn, dist_o, coarse_o, embed_o):
    total = jnp.sum(di[...])
    inv = 0.5 / total
    dist_o[...] = (di[0] + di[1] + do_[0] + do_[1]) * inv
    coarse_o[...] = co[0] + co[1]
    cnt = cn[0] + cn[1]
    embed_o[...] = (em[0] + em[1]) / jnp.clip(cnt, 1.0)[:, None]


def _build_calls():
    mesh = plsc.VectorSubcoreMesh(core_axis_name="c", subcore_axis_name="s",
                                  num_cores=NC, num_subcores=NS)

    stage_a = functools.partial(
        pl.kernel,
        out_type=(
            jax.ShapeDtypeStruct((NC * DEG_P,), f32),
            jax.ShapeDtypeStruct((NC * DEG_P,), f32),
            jax.ShapeDtypeStruct((NC * CC_P,), f32),
            jax.ShapeDtypeStruct((NC * R_P, D), f32),
            jax.ShapeDtypeStruct((NC * R_P,), f32),
            jax.ShapeDtypeStruct((NC * 256,), f32),
        ),
        mesh=mesh,
        compiler_params=pltpu.CompilerParams(needs_layout_passes=False),
        scratch_types=[
            pltpu.VMEM_SHARED((DEG_P,), f32),
            pltpu.VMEM_SHARED((DEG_P,), f32),
            pltpu.VMEM_SHARED((CC_P,), f32),
            pltpu.VMEM_SHARED((R_P, D), f32),
            pltpu.VMEM_SHARED((R_P,), f32),
            pltpu.VMEM_SHARED((256,), f32),
            pltpu.VMEM((N_PAD,), i32),
            pltpu.VMEM((NB, CHUNK), i32),
            pltpu.VMEM((NB, CHUNK), i32),
            pltpu.VMEM((NB, CHUNK), i32),
            pltpu.VMEM((NB, CHUNK), f32),
            pltpu.VMEM((CHUNK, D), f32),
            pltpu.VMEM((1, CHUNK), i32),
            pltpu.VMEM((CHUNK,), f32),
            pltpu.VMEM((640,), f32),
            pltpu.VMEM((L,), f32),
            pltpu.VMEM((15680,), f32),
            pltpu.VMEM((32, D), f32),
            pltpu.SemaphoreType.DMA,
            pltpu.SemaphoreType.DMA,
        ],
    )(_stage_a_body)

    stage_b = pl.pallas_call(
        _stage_b_tc,
        out_shape=(
            jax.ShapeDtypeStruct((DEG_P // 128, 128), f32),
            jax.ShapeDtypeStruct((CC_P // 128, 128), f32),
            jax.ShapeDtypeStruct((R_P, D), f32),
        ),
    )

    return stage_a, stage_b


_CALLS = []


@jax.jit
def kernel(x, edge_index, edge_weight, comms_idx):
    if not _CALLS:
        _CALLS.extend(_build_calls())
    _STAGE_A, _STAGE_B = _CALLS
    src = edge_index[0].astype(i32)
    dst = edge_index[1].astype(i32)
    w = edge_weight.astype(f32)
    comms = comms_idx.astype(i32)

    spread_e = (jnp.arange(E_PAD - E, dtype=i32) * 97) % N
    src_p = jnp.concatenate([src, spread_e])
    dst_p = jnp.concatenate([dst, spread_e])
    w_p = jnp.concatenate([w, jnp.zeros((E_PAD - E,), f32)])
    spread_n = (jnp.arange(N_PAD - N, dtype=i32) * 97) % C
    comms_p = jnp.concatenate([comms, spread_n])
    x_p = jnp.concatenate([x.astype(f32), jnp.zeros((N_PAD - N, D), f32)])
    ones_p = jnp.concatenate([jnp.ones((N,), f32), jnp.zeros((N_PAD - N,), f32)])

    deg_in2, deg_out2, coarse2, embed2, cnt2, tot2 = _STAGE_A(
        src_p, dst_p, w_p, comms_p, x_p, ones_p)
    del tot2
    dist_p, coarse_p, embed_p = _STAGE_B(
        deg_in2.reshape(NC, DEG_P // 128, 128),
        deg_out2.reshape(NC, DEG_P // 128, 128),
        coarse2.reshape(NC, CC_P // 128, 128),
        embed2.reshape(NC, R_P, D),
        cnt2.reshape(NC, R_P))

    embed = embed_p[:C]
    dist = dist_p.reshape(DEG_P)[:N]
    coarse_w = coarse_p.reshape(CC_P)[:CC].reshape(C, C)
    return embed, dist, coarse_w
